# baseline (speedup 1.0000x reference)
import functools

import jax
import jax.numpy as jnp
from jax import lax
from jax.experimental import pallas as pl
from jax.experimental.pallas import tpu as pltpu
from jax.experimental.pallas import tpu_sc as plsc

_B = 16384
_L = 200
_C = 5
_VPAD = 48
_NW = 32
_ROWS_W = _B // _NW
_CHUNK_ROWS = 16
_NCHUNK = _ROWS_W // _CHUNK_ROWS
_PAIRS = _CHUNK_ROWS // 2
_VECS = 2 * _L // 16

_mesh = plsc.VectorSubcoreMesh(core_axis_name="c", subcore_axis_name="s")


@functools.partial(
    pl.kernel,
    out_type=jax.ShapeDtypeStruct((_B, _C, _L), jnp.float32),
    mesh=_mesh,
    compiler_params=pltpu.CompilerParams(needs_layout_passes=False),
    scratch_types=[
        pltpu.VMEM((_VPAD,), jnp.float32),
        pltpu.VMEM((_VPAD,), jnp.float32),
        pltpu.VMEM((_VPAD,), jnp.float32),
        pltpu.VMEM((_VPAD,), jnp.float32),
        pltpu.VMEM((_VPAD,), jnp.float32),
        pltpu.VMEM((_CHUNK_ROWS, _L), jnp.int32),
        pltpu.VMEM((_CHUNK_ROWS, _L), jnp.int32),
        pltpu.VMEM((_CHUNK_ROWS, _C, _L), jnp.float32),
        pltpu.VMEM((_CHUNK_ROWS, _C, _L), jnp.float32),
        pltpu.SemaphoreType.DMA,
        pltpu.SemaphoreType.DMA,
        pltpu.SemaphoreType.DMA,
        pltpu.SemaphoreType.DMA,
    ],
)
def _lookup_kernel(xf, tabf, outf, t0, t1, t2, t3, t4, x_a, x_b, o_a, o_b,
                   sx_a, sx_b, so_a, so_b):
    cid = lax.axis_index("c")
    sid = lax.axis_index("s")
    wid = sid * 2 + cid

    tabs = (t0, t1, t2, t3, t4)
    for c in range(_C):
        pltpu.sync_copy(tabf.at[pl.ds(c * _VPAD, _VPAD)], tabs[c])

    row0w = wid * _ROWS_W

    def compute(x_v, o_v):
        def row_body(r, carry):
            pend = None
            for j in range(13):
                col = 16 * j if j < 12 else _L - 16
                xv = x_v[r, pl.ds(col, 16)]
                vals = [plsc.load_gather(tabs[c], [xv]) for c in range(_C)]
                if pend is not None:
                    pcol, pvals = pend
                    for c in range(_C):
                        o_v[r, c, pl.ds(pcol, 16)] = pvals[c]
                pend = (col, vals)
            pcol, pvals = pend
            for c in range(_C):
                o_v[r, c, pl.ds(pcol, 16)] = pvals[c]
            return carry
        lax.fori_loop(0, _CHUNK_ROWS, row_body, 0)

    xbufs = (x_a, x_b)
    obufs = (o_a, o_b)
    xsems = (sx_a, sx_b)
    osems = (so_a, so_b)

    for b in range(2):
        pltpu.async_copy(
            xf.at[pl.ds(row0w + b * _CHUNK_ROWS, _CHUNK_ROWS)],
            xbufs[b], xsems[b])

    def ring_body(k2, carry):
        for b in range(2):
            k = 2 * k2 + b
            row0 = row0w + k * _CHUNK_ROWS
            pltpu.make_async_copy(
                xf.at[pl.ds(row0, _CHUNK_ROWS)], xbufs[b], xsems[b]).wait()

            @pl.when(k2 > 0)
            def _():
                pltpu.make_async_copy(
                    obufs[b], outf.at[pl.ds(row0w, _CHUNK_ROWS)],
                    osems[b]).wait()

            compute(xbufs[b], obufs[b])
            pltpu.async_copy(
                obufs[b], outf.at[pl.ds(row0, _CHUNK_ROWS)], osems[b])

            @pl.when(k2 < _NCHUNK // 2 - 1)
            def _():
                pltpu.async_copy(
                    xf.at[pl.ds(row0 + 2 * _CHUNK_ROWS, _CHUNK_ROWS)],
                    xbufs[b], xsems[b])
        return carry

    lax.fori_loop(0, _NCHUNK // 2, ring_body, 0)

    for b in range(2):
        pltpu.make_async_copy(
            obufs[b], outf.at[pl.ds(row0w, _CHUNK_ROWS)], osems[b]).wait()


def kernel(x, map_table):
    tab = jnp.zeros((_C, _VPAD), jnp.float32).at[:, : map_table.shape[0]].set(
        map_table.T
    )
    return _lookup_kernel(x, tab.reshape(-1))
